# Initial kernel scaffold; baseline (speedup 1.0000x reference)
import functools

import jax
import jax.numpy as jnp
import numpy as np
from jax import lax
from jax.experimental import pallas as pl
from jax.experimental.pallas import tpu as pltpu
from jax.experimental.pallas import tpu_sc as plsc

_X32 = np.float32(1.0 - 1e-5)
_TAU = float(np.float32(0.5) * np.log(np.float32(1.0 + _X32) / np.float32(1.0 - _X32)))
_MAXNORM = float(_X32)

_HALF = 16
_CH = 1024
_IB = 128
_NT = 16
_NG = 64


def _clip_scale(sq):
    n = jnp.maximum(jnp.sqrt(sq), 1e-15)
    return jnp.minimum(n, _TAU) / n


def _first_body(x_ref, w_ref, t0_ref, t1_ref):
    x = x_ref[...]
    u = x * _clip_scale(jnp.sum(x * x, -1, keepdims=True))
    v = lax.dot_general(u, w_ref[...], (((1,), (1,)), ((), ())),
                        preferred_element_type=jnp.float32)
    t = v * _clip_scale(jnp.sum(v * v, -1, keepdims=True))
    t0_ref[...] = t[:, :_HALF]
    t1_ref[...] = t[:, _HALF:]


def _first_layer(x, w0, bn):
    n = x.shape[0]
    return pl.pallas_call(
        _first_body,
        grid=(n // bn,),
        in_specs=[
            pl.BlockSpec((bn, x.shape[1]), lambda i: (i, 0)),
            pl.BlockSpec(w0.shape, lambda i: (0, 0)),
        ],
        out_specs=[
            pl.BlockSpec((bn, _HALF), lambda i: (i, 0)),
            pl.BlockSpec((bn, _HALF), lambda i: (i, 0)),
        ],
        out_shape=[
            jax.ShapeDtypeStruct((n, _HALF), jnp.float32),
            jax.ShapeDtypeStruct((n, _HALF), jnp.float32),
        ],
    )(x, w0)


def _mid_body(a0_ref, a1_ref, d0_ref, d1_ref, w_ref, t0_ref, t1_ref):
    inv = 1.0 / jnp.maximum(d0_ref[:, :1] + d1_ref[:, :1], 1.0)
    a0 = a0_ref[...] * inv
    a1 = a1_ref[...] * inv
    s = _clip_scale(jnp.sum(a0 * a0, -1, keepdims=True)
                    + jnp.sum(a1 * a1, -1, keepdims=True))
    u0 = jnp.maximum(a0 * s, 0.0)
    u1 = jnp.maximum(a1 * s, 0.0)
    w = w_ref[...]
    v = (lax.dot_general(u0, w[:, :_HALF], (((1,), (1,)), ((), ())),
                         preferred_element_type=jnp.float32)
         + lax.dot_general(u1, w[:, _HALF:], (((1,), (1,)), ((), ())),
                           preferred_element_type=jnp.float32))
    t = v * _clip_scale(jnp.sum(v * v, -1, keepdims=True))
    t0_ref[...] = t[:, :_HALF]
    t1_ref[...] = t[:, _HALF:]


def _mid_layer(a0, a1, d0, d1, w, n, bn):
    return pl.pallas_call(
        _mid_body,
        grid=(n // bn,),
        in_specs=[
            pl.BlockSpec((bn, _HALF), lambda i: (i, 0)),
            pl.BlockSpec((bn, _HALF), lambda i: (i, 0)),
            pl.BlockSpec((bn, _HALF), lambda i: (i, 0)),
            pl.BlockSpec((bn, _HALF), lambda i: (i, 0)),
            pl.BlockSpec(w.shape, lambda i: (0, 0)),
        ],
        out_specs=[
            pl.BlockSpec((bn, _HALF), lambda i: (i, 0)),
            pl.BlockSpec((bn, _HALF), lambda i: (i, 0)),
        ],
        out_shape=[
            jax.ShapeDtypeStruct((n, _HALF), jnp.float32),
            jax.ShapeDtypeStruct((n, _HALF), jnp.float32),
        ],
    )(a0, a1, d0, d1, w)


def _final_body(a0_ref, a1_ref, d0_ref, d1_ref, b_ref, wc_ref, o_ref,
                sum_s, max_s, cnt_s):
    i = pl.program_id(0)
    nsteps = pl.num_programs(0)

    @pl.when(i == 0)
    def _():
        sum_s[...] = jnp.zeros_like(sum_s)
        max_s[...] = jnp.zeros_like(max_s)
        cnt_s[...] = jnp.zeros_like(cnt_s)

    inv = 1.0 / jnp.maximum(d0_ref[:, :1] + d1_ref[:, :1], 1.0)
    a0 = a0_ref[...] * inv
    a1 = a1_ref[...] * inv
    s = _clip_scale(jnp.sum(a0 * a0, -1, keepdims=True)
                    + jnp.sum(a1 * a1, -1, keepdims=True))
    u0 = jnp.maximum(a0 * s, 0.0)
    u1 = jnp.maximum(a1 * s, 0.0)
    nrm = jnp.maximum(jnp.sqrt(jnp.sum(u0 * u0, -1, keepdims=True)
                               + jnp.sum(u1 * u1, -1, keepdims=True)), 1e-15)
    hs = jnp.minimum(jnp.tanh(nrm), _MAXNORM) / nrm
    h = jnp.concatenate([u0 * hs, u1 * hs], axis=1)

    seg = lax.broadcasted_iota(jnp.int32, (1, _NG), 1)
    mask = (b_ref[...] == seg).astype(jnp.float32)
    sum_s[...] += lax.dot_general(mask, h, (((0,), (0,)), ((), ())),
                                  preferred_element_type=jnp.float32)
    cnt_s[...] += lax.dot_general(
        mask, jnp.ones((mask.shape[0], 1), jnp.float32),
        (((0,), (0,)), ((), ())), preferred_element_type=jnp.float32)
    blockmax = jnp.max(mask[:, :, None] * h[:, None, :], axis=0)
    max_s[...] = jnp.maximum(max_s[...], blockmax)

    @pl.when(i == nsteps - 1)
    def _():
        gap = sum_s[...] / jnp.maximum(cnt_s[...], 1.0)
        pooled = jnp.concatenate([gap, max_s[...]], axis=1)
        o_ref[...] = lax.dot_general(pooled, wc_ref[...],
                                     (((1,), (0,)), ((), ())),
                                     preferred_element_type=jnp.float32)


def _final_layer(a0, a1, d0, d1, batch2d, wct, n, bn):
    hid2 = 2 * _HALF
    return pl.pallas_call(
        _final_body,
        grid=(n // bn,),
        in_specs=[
            pl.BlockSpec((bn, _HALF), lambda i: (i, 0)),
            pl.BlockSpec((bn, _HALF), lambda i: (i, 0)),
            pl.BlockSpec((bn, _HALF), lambda i: (i, 0)),
            pl.BlockSpec((bn, _HALF), lambda i: (i, 0)),
            pl.BlockSpec((bn, 1), lambda i: (i, 0)),
            pl.BlockSpec(wct.shape, lambda i: (0, 0)),
        ],
        out_specs=pl.BlockSpec((_NG, 1), lambda i: (0, 0)),
        out_shape=jax.ShapeDtypeStruct((_NG, 1), jnp.float32),
        scratch_shapes=[
            pltpu.VMEM((_NG, hid2), jnp.float32),
            pltpu.VMEM((_NG, hid2), jnp.float32),
            pltpu.VMEM((_NG, 1), jnp.float32),
        ],
    )(a0, a1, d0, d1, batch2d, wct)


def _fill_doubling(ref, nrows):
    k = 1
    while k < nrows:
        sz = min(k, nrows - k)
        pltpu.sync_copy(ref.at[pl.ds(0, sz)], ref.at[pl.ds(k, sz)])
        k += k


def _zero_acc_slice(acc, zbuf, base, rows):
    off = 0
    while off < rows:
        sz = min(zbuf.shape[0], rows - off)
        pltpu.sync_copy(zbuf.at[pl.ds(0, sz)], acc.at[pl.ds(base + off, sz)])
        off += sz


def _make_agg_kernel(n_acc, rows_per_tile, chunks_per_tile):
    mesh = plsc.VectorSubcoreMesh(core_axis_name="c", subcore_axis_name="s")
    rpc = _CH // _IB

    def body(t0_hbm, t1_hbm, src_hbm, dst_hbm, out0, out1,
             acc, zbuf, sidx, didx, rows, gsem):
        c = lax.axis_index("c")
        s = lax.axis_index("s")
        base = s * rows_per_tile

        zbuf[0, :] = jnp.zeros((16,), jnp.float32)
        _fill_doubling(zbuf, zbuf.shape[0])
        _zero_acc_slice(acc, zbuf, base, rows_per_tile)
        plsc.subcore_barrier()

        def chunk(i, carry):
            r0 = s * (chunks_per_tile * rpc) + i * rpc
            pltpu.sync_copy(src_hbm.at[pl.ds(r0, rpc)], sidx)
            pltpu.sync_copy(dst_hbm.at[pl.ds(r0, rpc)], didx)

            @pl.when(c == 0)
            def _():
                descs = [pltpu.async_copy(t0_hbm.at[sidx.at[j]], rows.at[j], gsem)
                         for j in range(rpc)]
                for d in descs:
                    d.wait()

            @pl.when(c == 1)
            def _():
                descs = [pltpu.async_copy(t1_hbm.at[sidx.at[j]], rows.at[j], gsem)
                         for j in range(rpc)]
                for d in descs:
                    d.wait()

            for j in range(rpc):
                pltpu.sync_copy(rows.at[j], acc.at[didx.at[j]], add=True)
            return carry

        lax.fori_loop(0, chunks_per_tile, chunk, 0)
        plsc.subcore_barrier()

        @pl.when(c == 0)
        def _():
            pltpu.sync_copy(acc.at[pl.ds(base, rows_per_tile)],
                            out0.at[pl.ds(base, rows_per_tile)])

        @pl.when(c == 1)
        def _():
            pltpu.sync_copy(acc.at[pl.ds(base, rows_per_tile)],
                            out1.at[pl.ds(base, rows_per_tile)])

    return pl.kernel(
        body,
        out_type=[
            jax.ShapeDtypeStruct((n_acc, _HALF), jnp.float32),
            jax.ShapeDtypeStruct((n_acc, _HALF), jnp.float32),
        ],
        mesh=mesh,
        scratch_types=[
            pltpu.VMEM_SHARED((n_acc, _HALF), jnp.float32),
            pltpu.VMEM((1024, 16), jnp.float32),
            pltpu.VMEM((rpc, _IB), jnp.int32),
            pltpu.VMEM((rpc, _IB), jnp.int32),
            pltpu.VMEM((rpc, _IB, _HALF), jnp.float32),
            pltpu.SemaphoreType.DMA,
        ],
    )


def _make_deg_kernel(n_acc, rows_per_tile, chunks_per_range):
    mesh = plsc.VectorSubcoreMesh(core_axis_name="c", subcore_axis_name="s")
    rpc = _CH // _IB

    def body(dst_hbm, out0, out1, acc, zbuf, ones_v, didx):
        c = lax.axis_index("c")
        s = lax.axis_index("s")
        base = s * rows_per_tile

        zbuf[0, :] = jnp.zeros((16,), jnp.float32)
        _fill_doubling(zbuf, zbuf.shape[0])
        _zero_acc_slice(acc, zbuf, base, rows_per_tile)
        ones_v[0, :] = jnp.full((16,), 1.0, jnp.float32)
        _fill_doubling(ones_v, ones_v.shape[0])
        plsc.subcore_barrier()

        def chunk(i, carry):
            w = c * _NT + s
            r0 = w * (chunks_per_range * rpc) + i * rpc
            pltpu.sync_copy(dst_hbm.at[pl.ds(r0, rpc)], didx)
            for j in range(rpc):
                pltpu.sync_copy(ones_v, acc.at[didx.at[j]], add=True)
            return carry

        lax.fori_loop(0, chunks_per_range, chunk, 0)
        plsc.subcore_barrier()

        @pl.when(c == 0)
        def _():
            pltpu.sync_copy(acc.at[pl.ds(base, rows_per_tile)],
                            out0.at[pl.ds(base, rows_per_tile)])

        @pl.when(c == 1)
        def _():
            pltpu.sync_copy(acc.at[pl.ds(base, rows_per_tile)],
                            out1.at[pl.ds(base, rows_per_tile)])

    return pl.kernel(
        body,
        out_type=[
            jax.ShapeDtypeStruct((n_acc, _HALF), jnp.float32),
            jax.ShapeDtypeStruct((n_acc, _HALF), jnp.float32),
        ],
        mesh=mesh,
        scratch_types=[
            pltpu.VMEM_SHARED((n_acc, _HALF), jnp.float32),
            pltpu.VMEM((1024, 16), jnp.float32),
            pltpu.VMEM((_IB, _HALF), jnp.float32),
            pltpu.VMEM((rpc, _IB), jnp.int32),
        ],
    )


def kernel(x, edge_index, batch, W0, Ws, bs, Wc, bc):
    del bs, bc
    n = x.shape[0]
    e = edge_index.shape[1]

    rows_per_tile = (((n + _NT - 1) // _NT) + 7) // 8 * 8
    n_acc = rows_per_tile * _NT
    chunks_per_tile = (e + _NT * _CH - 1) // (_NT * _CH)
    e_pad = _NT * _CH * chunks_per_tile
    chunks_per_range = chunks_per_tile // 2

    src = edge_index[0]
    dst = edge_index[1]
    pad = e_pad - e
    if pad:
        src = jnp.concatenate([src, jnp.zeros((pad,), jnp.int32)])
        dst = jnp.concatenate([dst, jnp.full((pad,), n, jnp.int32)])
    src2d = src.reshape(-1, _IB)
    dst2d = dst.reshape(-1, _IB)

    agg = _make_agg_kernel(n_acc, rows_per_tile, chunks_per_tile)
    degk = _make_deg_kernel(n_acc, rows_per_tile, chunks_per_range)

    dg0, dg1 = degk(dst2d)
    d0 = dg0[:n, :]
    d1 = dg1[:n, :]

    bn = 10000
    t0, t1 = _first_layer(x, W0, bn)
    n_layers = Ws.shape[0] + 1
    a0 = a1 = None
    for i in range(n_layers):
        a0p, a1p = agg(t0, t1, src2d, dst2d)
        a0 = a0p[:n, :]
        a1 = a1p[:n, :]
        if i + 1 < n_layers:
            t0, t1 = _mid_layer(a0, a1, d0, d1, Ws[i], n, bn)

    batch2d = batch.reshape(n, 1)
    wct = Wc.reshape(-1, 1)
    return _final_layer(a0, a1, d0, d1, batch2d, wct, n, 1000)


# speedup vs baseline: 6.9590x; 6.9590x over previous
import functools

import jax
import jax.numpy as jnp
import numpy as np
from jax import lax
from jax.experimental import pallas as pl
from jax.experimental.pallas import tpu as pltpu
from jax.experimental.pallas import tpu_sc as plsc

_X32 = np.float32(1.0 - 1e-5)
_TAU = float(np.float32(0.5) * np.log(np.float32(1.0 + _X32) / np.float32(1.0 - _X32)))
_MAXNORM = float(_X32)

_HALF = 16
_CH = 512
_IB = 128
_NT = 16
_NG = 64


def _clip_scale(sq):
    n = jnp.maximum(jnp.sqrt(sq), 1e-15)
    return jnp.minimum(n, _TAU) / n


def _first_body(x_ref, w_ref, t0_ref, t1_ref):
    x = x_ref[...]
    u = x * _clip_scale(jnp.sum(x * x, -1, keepdims=True))
    v = lax.dot_general(u, w_ref[...], (((1,), (1,)), ((), ())),
                        preferred_element_type=jnp.float32)
    t = v * _clip_scale(jnp.sum(v * v, -1, keepdims=True))
    t0_ref[...] = t[:, :_HALF]
    t1_ref[...] = t[:, _HALF:]


def _first_layer(x, w0, bn):
    n = x.shape[0]
    return pl.pallas_call(
        _first_body,
        grid=(n // bn,),
        in_specs=[
            pl.BlockSpec((bn, x.shape[1]), lambda i: (i, 0)),
            pl.BlockSpec(w0.shape, lambda i: (0, 0)),
        ],
        out_specs=[
            pl.BlockSpec((bn, _HALF), lambda i: (i, 0)),
            pl.BlockSpec((bn, _HALF), lambda i: (i, 0)),
        ],
        out_shape=[
            jax.ShapeDtypeStruct((n, _HALF), jnp.float32),
            jax.ShapeDtypeStruct((n, _HALF), jnp.float32),
        ],
    )(x, w0)


def _mid_body(a0_ref, a1_ref, d0_ref, d1_ref, w_ref, t0_ref, t1_ref):
    inv = 1.0 / jnp.maximum(d0_ref[:, :1] + d1_ref[:, :1], 1.0)
    a0 = a0_ref[...] * inv
    a1 = a1_ref[...] * inv
    s = _clip_scale(jnp.sum(a0 * a0, -1, keepdims=True)
                    + jnp.sum(a1 * a1, -1, keepdims=True))
    u0 = jnp.maximum(a0 * s, 0.0)
    u1 = jnp.maximum(a1 * s, 0.0)
    w = w_ref[...]
    v = (lax.dot_general(u0, w[:, :_HALF], (((1,), (1,)), ((), ())),
                         preferred_element_type=jnp.float32)
         + lax.dot_general(u1, w[:, _HALF:], (((1,), (1,)), ((), ())),
                           preferred_element_type=jnp.float32))
    t = v * _clip_scale(jnp.sum(v * v, -1, keepdims=True))
    t0_ref[...] = t[:, :_HALF]
    t1_ref[...] = t[:, _HALF:]


def _mid_layer(a0, a1, d0, d1, w, n, bn):
    return pl.pallas_call(
        _mid_body,
        grid=(n // bn,),
        in_specs=[
            pl.BlockSpec((bn, _HALF), lambda i: (i, 0)),
            pl.BlockSpec((bn, _HALF), lambda i: (i, 0)),
            pl.BlockSpec((bn, _HALF), lambda i: (i, 0)),
            pl.BlockSpec((bn, _HALF), lambda i: (i, 0)),
            pl.BlockSpec(w.shape, lambda i: (0, 0)),
        ],
        out_specs=[
            pl.BlockSpec((bn, _HALF), lambda i: (i, 0)),
            pl.BlockSpec((bn, _HALF), lambda i: (i, 0)),
        ],
        out_shape=[
            jax.ShapeDtypeStruct((n, _HALF), jnp.float32),
            jax.ShapeDtypeStruct((n, _HALF), jnp.float32),
        ],
    )(a0, a1, d0, d1, w)


def _final_body(a0_ref, a1_ref, d0_ref, d1_ref, b_ref, wc_ref, o_ref,
                sum_s, max_s, cnt_s):
    i = pl.program_id(0)
    nsteps = pl.num_programs(0)

    @pl.when(i == 0)
    def _():
        sum_s[...] = jnp.zeros_like(sum_s)
        max_s[...] = jnp.zeros_like(max_s)
        cnt_s[...] = jnp.zeros_like(cnt_s)

    inv = 1.0 / jnp.maximum(d0_ref[:, :1] + d1_ref[:, :1], 1.0)
    a0 = a0_ref[...] * inv
    a1 = a1_ref[...] * inv
    s = _clip_scale(jnp.sum(a0 * a0, -1, keepdims=True)
                    + jnp.sum(a1 * a1, -1, keepdims=True))
    u0 = jnp.maximum(a0 * s, 0.0)
    u1 = jnp.maximum(a1 * s, 0.0)
    nrm = jnp.maximum(jnp.sqrt(jnp.sum(u0 * u0, -1, keepdims=True)
                               + jnp.sum(u1 * u1, -1, keepdims=True)), 1e-15)
    hs = jnp.minimum(jnp.tanh(nrm), _MAXNORM) / nrm
    h = jnp.concatenate([u0 * hs, u1 * hs], axis=1)

    seg = lax.broadcasted_iota(jnp.int32, (1, _NG), 1)
    mask = (b_ref[...] == seg).astype(jnp.float32)
    sum_s[...] += lax.dot_general(mask, h, (((0,), (0,)), ((), ())),
                                  preferred_element_type=jnp.float32)
    cnt_s[...] += lax.dot_general(
        mask, jnp.ones((mask.shape[0], 1), jnp.float32),
        (((0,), (0,)), ((), ())), preferred_element_type=jnp.float32)
    blockmax = jnp.concatenate(
        [jnp.max(h * mask[:, g:g + 1], axis=0, keepdims=True)
         for g in range(_NG)], axis=0)
    max_s[...] = jnp.maximum(max_s[...], blockmax)

    @pl.when(i == nsteps - 1)
    def _():
        gap = sum_s[...] / jnp.maximum(cnt_s[...], 1.0)
        pooled = jnp.concatenate([gap, max_s[...]], axis=1)
        o_ref[...] = lax.dot_general(pooled, wc_ref[...],
                                     (((1,), (0,)), ((), ())),
                                     preferred_element_type=jnp.float32)


def _final_layer(a0, a1, d0, d1, batch2d, wct, n, bn):
    hid2 = 2 * _HALF
    return pl.pallas_call(
        _final_body,
        grid=(n // bn,),
        in_specs=[
            pl.BlockSpec((bn, _HALF), lambda i: (i, 0)),
            pl.BlockSpec((bn, _HALF), lambda i: (i, 0)),
            pl.BlockSpec((bn, _HALF), lambda i: (i, 0)),
            pl.BlockSpec((bn, _HALF), lambda i: (i, 0)),
            pl.BlockSpec((bn, 1), lambda i: (i, 0)),
            pl.BlockSpec(wct.shape, lambda i: (0, 0)),
        ],
        out_specs=pl.BlockSpec((_NG, 1), lambda i: (0, 0)),
        out_shape=jax.ShapeDtypeStruct((_NG, 1), jnp.float32),
        scratch_shapes=[
            pltpu.VMEM((_NG, hid2), jnp.float32),
            pltpu.VMEM((_NG, hid2), jnp.float32),
            pltpu.VMEM((_NG, 1), jnp.float32),
        ],
    )(a0, a1, d0, d1, batch2d, wct)


def _zero_acc_slice(acc, zbuf, base, rows):
    off = 0
    while off < rows:
        sz = min(zbuf.shape[0], rows - off)
        pltpu.sync_copy(zbuf.at[pl.ds(0, sz)], acc.at[pl.ds(base + off, sz)])
        off += sz


def _make_agg_kernel(n_acc, rows_per_tile, chunks_per_tile):
    mesh = plsc.VectorSubcoreMesh(core_axis_name="c", subcore_axis_name="s")
    rpc = _CH // _IB

    def body(t0_hbm, t1_hbm, src_hbm, dst_hbm, z_hbm, out0, out1,
             acc, zbuf, sidx, didx, rows, gsem):
        c = lax.axis_index("c")
        s = lax.axis_index("s")
        base = s * rows_per_tile

        pltpu.sync_copy(z_hbm, zbuf)
        _zero_acc_slice(acc, zbuf, base, rows_per_tile)
        plsc.subcore_barrier()

        def chunk(i, carry):
            r0 = s * (chunks_per_tile * rpc) + i * rpc
            pltpu.sync_copy(src_hbm.at[pl.ds(r0, rpc)], sidx)
            pltpu.sync_copy(dst_hbm.at[pl.ds(r0, rpc)], didx)

            @pl.when(c == 0)
            def _():
                descs = [pltpu.async_copy(t0_hbm.at[sidx.at[j]], rows.at[j], gsem)
                         for j in range(rpc)]
                for d in descs:
                    d.wait()

            @pl.when(c == 1)
            def _():
                descs = [pltpu.async_copy(t1_hbm.at[sidx.at[j]], rows.at[j], gsem)
                         for j in range(rpc)]
                for d in descs:
                    d.wait()

            for j in range(rpc):
                pltpu.sync_copy(rows.at[j], acc.at[didx.at[j]], add=True)
            return carry

        lax.fori_loop(0, chunks_per_tile, chunk, 0)
        plsc.subcore_barrier()

        @pl.when(c == 0)
        def _():
            pltpu.sync_copy(acc.at[pl.ds(base, rows_per_tile)],
                            out0.at[pl.ds(base, rows_per_tile)])

        @pl.when(c == 1)
        def _():
            pltpu.sync_copy(acc.at[pl.ds(base, rows_per_tile)],
                            out1.at[pl.ds(base, rows_per_tile)])

    return pl.kernel(
        body,
        out_type=[
            jax.ShapeDtypeStruct((n_acc, _HALF), jnp.float32),
            jax.ShapeDtypeStruct((n_acc, _HALF), jnp.float32),
        ],
        mesh=mesh,
        scratch_types=[
            pltpu.VMEM_SHARED((n_acc, _HALF), jnp.float32),
            pltpu.VMEM((512, 16), jnp.float32),
            pltpu.VMEM((rpc, _IB), jnp.int32),
            pltpu.VMEM((rpc, _IB), jnp.int32),
            pltpu.VMEM((rpc, _IB, _HALF), jnp.float32),
            pltpu.SemaphoreType.DMA,
        ],
        compiler_params=pltpu.CompilerParams(use_tc_tiling_on_sc=False),
    )


def _make_deg_kernel(n_acc, rows_per_tile, chunks_per_range):
    mesh = plsc.VectorSubcoreMesh(core_axis_name="c", subcore_axis_name="s")
    rpc = _CH // _IB

    def body(dst_hbm, z_hbm, ones_hbm, out0, out1, acc, zbuf, ones_v, didx):
        c = lax.axis_index("c")
        s = lax.axis_index("s")
        base = s * rows_per_tile

        pltpu.sync_copy(z_hbm, zbuf)
        _zero_acc_slice(acc, zbuf, base, rows_per_tile)
        pltpu.sync_copy(ones_hbm, ones_v)
        plsc.subcore_barrier()

        def chunk(i, carry):
            w = c * _NT + s
            r0 = w * (chunks_per_range * rpc) + i * rpc
            pltpu.sync_copy(dst_hbm.at[pl.ds(r0, rpc)], didx)
            for j in range(rpc):
                pltpu.sync_copy(ones_v, acc.at[didx.at[j]], add=True)
            return carry

        lax.fori_loop(0, chunks_per_range, chunk, 0)
        plsc.subcore_barrier()

        @pl.when(c == 0)
        def _():
            pltpu.sync_copy(acc.at[pl.ds(base, rows_per_tile)],
                            out0.at[pl.ds(base, rows_per_tile)])

        @pl.when(c == 1)
        def _():
            pltpu.sync_copy(acc.at[pl.ds(base, rows_per_tile)],
                            out1.at[pl.ds(base, rows_per_tile)])

    return pl.kernel(
        body,
        out_type=[
            jax.ShapeDtypeStruct((n_acc, _HALF), jnp.float32),
            jax.ShapeDtypeStruct((n_acc, _HALF), jnp.float32),
        ],
        mesh=mesh,
        scratch_types=[
            pltpu.VMEM_SHARED((n_acc, _HALF), jnp.float32),
            pltpu.VMEM((512, 16), jnp.float32),
            pltpu.VMEM((_IB, _HALF), jnp.float32),
            pltpu.VMEM((rpc, _IB), jnp.int32),
        ],
        compiler_params=pltpu.CompilerParams(use_tc_tiling_on_sc=False),
    )


def kernel(x, edge_index, batch, W0, Ws, bs, Wc, bc):
    del bs, bc
    n = x.shape[0]
    e = edge_index.shape[1]

    rows_per_tile = (((n + _NT - 1) // _NT) + 7) // 8 * 8
    n_acc = rows_per_tile * _NT
    chunks_per_tile = (e + _NT * _CH - 1) // (_NT * _CH)
    e_pad = _NT * _CH * chunks_per_tile
    chunks_per_range = chunks_per_tile // 2

    src = edge_index[0]
    dst = edge_index[1]
    pad = e_pad - e
    if pad:
        src = jnp.concatenate([src, jnp.zeros((pad,), jnp.int32)])
        dst = jnp.concatenate([dst, jnp.full((pad,), n, jnp.int32)])
    src2d = src.reshape(-1, _IB)
    dst2d = dst.reshape(-1, _IB)

    agg = _make_agg_kernel(n_acc, rows_per_tile, chunks_per_tile)
    degk = _make_deg_kernel(n_acc, rows_per_tile, chunks_per_range)

    zeros2d = jnp.zeros((512, 16), jnp.float32)
    ones2d = jnp.ones((_IB, _HALF), jnp.float32)

    dg0, dg1 = degk(dst2d, zeros2d, ones2d)
    d0 = dg0[:n, :]
    d1 = dg1[:n, :]

    bn = 2000
    t0, t1 = _first_layer(x, W0, bn)
    n_layers = Ws.shape[0] + 1
    a0 = a1 = None
    for i in range(n_layers):
        a0p, a1p = agg(t0, t1, src2d, dst2d, zeros2d)
        a0 = a0p[:n, :]
        a1 = a1p[:n, :]
        if i + 1 < n_layers:
            t0, t1 = _mid_layer(a0, a1, d0, d1, Ws[i], n, bn)

    batch2d = batch.reshape(n, 1)
    wct = Wc.reshape(-1, 1)
    return _final_layer(a0, a1, d0, d1, batch2d, wct, n, 1000)


# speedup vs baseline: 10.8828x; 1.5638x over previous
import functools

import jax
import jax.numpy as jnp
import numpy as np
from jax import lax
from jax.experimental import pallas as pl
from jax.experimental.pallas import tpu as pltpu
from jax.experimental.pallas import tpu_sc as plsc

_X32 = np.float32(1.0 - 1e-5)
_TAU = float(np.float32(0.5) * np.log(np.float32(1.0 + _X32) / np.float32(1.0 - _X32)))
_MAXNORM = float(_X32)

_HALF = 16
_CH = 512
_IB = 128
_NT = 16
_NG = 64


def _clip_scale(sq):
    n = jnp.maximum(jnp.sqrt(sq), 1e-15)
    return jnp.minimum(n, _TAU) / n


def _first_body(x_ref, w_ref, t0_ref, t1_ref):
    x = x_ref[...]
    u = x * _clip_scale(jnp.sum(x * x, -1, keepdims=True))
    v = lax.dot_general(u, w_ref[...], (((1,), (1,)), ((), ())),
                        preferred_element_type=jnp.float32)
    t = v * _clip_scale(jnp.sum(v * v, -1, keepdims=True))
    t0_ref[...] = t[:, :_HALF]
    t1_ref[...] = t[:, _HALF:]


def _first_layer(x, w0, bn, n_acc):
    n = x.shape[0]
    return pl.pallas_call(
        _first_body,
        grid=(n // bn,),
        in_specs=[
            pl.BlockSpec((bn, x.shape[1]), lambda i: (i, 0)),
            pl.BlockSpec(w0.shape, lambda i: (0, 0)),
        ],
        out_specs=[
            pl.BlockSpec((bn, _HALF), lambda i: (i, 0)),
            pl.BlockSpec((bn, _HALF), lambda i: (i, 0)),
        ],
        out_shape=[
            jax.ShapeDtypeStruct((n_acc, _HALF), jnp.float32),
            jax.ShapeDtypeStruct((n_acc, _HALF), jnp.float32),
        ],
    )(x, w0)


def _mid_body(a0_ref, a1_ref, d0_ref, d1_ref, w_ref, t0_ref, t1_ref):
    inv = 1.0 / jnp.maximum(d0_ref[:, :1] + d1_ref[:, :1], 1.0)
    a0 = a0_ref[...] * inv
    a1 = a1_ref[...] * inv
    s = _clip_scale(jnp.sum(a0 * a0, -1, keepdims=True)
                    + jnp.sum(a1 * a1, -1, keepdims=True))
    u0 = jnp.maximum(a0 * s, 0.0)
    u1 = jnp.maximum(a1 * s, 0.0)
    w = w_ref[...]
    v = (lax.dot_general(u0, w[:, :_HALF], (((1,), (1,)), ((), ())),
                         preferred_element_type=jnp.float32)
         + lax.dot_general(u1, w[:, _HALF:], (((1,), (1,)), ((), ())),
                           preferred_element_type=jnp.float32))
    t = v * _clip_scale(jnp.sum(v * v, -1, keepdims=True))
    t0_ref[...] = t[:, :_HALF]
    t1_ref[...] = t[:, _HALF:]


def _mid_layer(a0, a1, d0, d1, w, n, bn, n_acc):
    return pl.pallas_call(
        _mid_body,
        grid=(n // bn,),
        in_specs=[
            pl.BlockSpec((bn, _HALF), lambda i: (i, 0)),
            pl.BlockSpec((bn, _HALF), lambda i: (i, 0)),
            pl.BlockSpec((bn, _HALF), lambda i: (i, 0)),
            pl.BlockSpec((bn, _HALF), lambda i: (i, 0)),
            pl.BlockSpec(w.shape, lambda i: (0, 0)),
        ],
        out_specs=[
            pl.BlockSpec((bn, _HALF), lambda i: (i, 0)),
            pl.BlockSpec((bn, _HALF), lambda i: (i, 0)),
        ],
        out_shape=[
            jax.ShapeDtypeStruct((n_acc, _HALF), jnp.float32),
            jax.ShapeDtypeStruct((n_acc, _HALF), jnp.float32),
        ],
    )(a0, a1, d0, d1, w)


def _final_body(a0_ref, a1_ref, d0_ref, d1_ref, b_ref, wc_ref, o_ref,
                sum_s, max_s, cnt_s):
    i = pl.program_id(0)
    nsteps = pl.num_programs(0)

    @pl.when(i == 0)
    def _():
        sum_s[...] = jnp.zeros_like(sum_s)
        max_s[...] = jnp.zeros_like(max_s)
        cnt_s[...] = jnp.zeros_like(cnt_s)

    inv = 1.0 / jnp.maximum(d0_ref[:, :1] + d1_ref[:, :1], 1.0)
    a0 = a0_ref[...] * inv
    a1 = a1_ref[...] * inv
    s = _clip_scale(jnp.sum(a0 * a0, -1, keepdims=True)
                    + jnp.sum(a1 * a1, -1, keepdims=True))
    u0 = jnp.maximum(a0 * s, 0.0)
    u1 = jnp.maximum(a1 * s, 0.0)
    nrm = jnp.maximum(jnp.sqrt(jnp.sum(u0 * u0, -1, keepdims=True)
                               + jnp.sum(u1 * u1, -1, keepdims=True)), 1e-15)
    hs = jnp.minimum(jnp.tanh(nrm), _MAXNORM) / nrm
    h = jnp.concatenate([u0 * hs, u1 * hs], axis=1)

    seg = lax.broadcasted_iota(jnp.int32, (1, _NG), 1)
    mask = (b_ref[...] == seg).astype(jnp.float32)
    sum_s[...] += lax.dot_general(mask, h, (((0,), (0,)), ((), ())),
                                  preferred_element_type=jnp.float32)
    cnt_s[...] += lax.dot_general(
        mask, jnp.ones((mask.shape[0], 1), jnp.float32),
        (((0,), (0,)), ((), ())), preferred_element_type=jnp.float32)
    blockmax = jnp.concatenate(
        [jnp.max(h * mask[:, g:g + 1], axis=0, keepdims=True)
         for g in range(_NG)], axis=0)
    max_s[...] = jnp.maximum(max_s[...], blockmax)

    @pl.when(i == nsteps - 1)
    def _():
        gap = sum_s[...] / jnp.maximum(cnt_s[...], 1.0)
        pooled = jnp.concatenate([gap, max_s[...]], axis=1)
        o_ref[...] = lax.dot_general(pooled, wc_ref[...],
                                     (((1,), (0,)), ((), ())),
                                     preferred_element_type=jnp.float32)


def _final_layer(a0, a1, d0, d1, batch2d, wct, n, bn):
    hid2 = 2 * _HALF
    return pl.pallas_call(
        _final_body,
        grid=(n // bn,),
        in_specs=[
            pl.BlockSpec((bn, _HALF), lambda i: (i, 0)),
            pl.BlockSpec((bn, _HALF), lambda i: (i, 0)),
            pl.BlockSpec((bn, _HALF), lambda i: (i, 0)),
            pl.BlockSpec((bn, _HALF), lambda i: (i, 0)),
            pl.BlockSpec((bn, 1), lambda i: (i, 0)),
            pl.BlockSpec(wct.shape, lambda i: (0, 0)),
        ],
        out_specs=pl.BlockSpec((_NG, 1), lambda i: (0, 0)),
        out_shape=jax.ShapeDtypeStruct((_NG, 1), jnp.float32),
        scratch_shapes=[
            pltpu.VMEM((_NG, hid2), jnp.float32),
            pltpu.VMEM((_NG, hid2), jnp.float32),
            pltpu.VMEM((_NG, 1), jnp.float32),
        ],
    )(a0, a1, d0, d1, batch2d, wct)


def _zero_acc_slice(acc, zbuf, base, rows):
    off = 0
    while off < rows:
        sz = min(zbuf.shape[0], rows - off)
        pltpu.sync_copy(zbuf.at[pl.ds(0, sz)], acc.at[pl.ds(base + off, sz)])
        off += sz


def _make_agg_kernel(n_acc, rows_per_tile, chunks_per_tile):
    mesh = plsc.VectorSubcoreMesh(core_axis_name="c", subcore_axis_name="s")
    rpc = _CH // _IB
    npairs = chunks_per_tile // 2

    def body(t0_hbm, t1_hbm, src_hbm, dst_hbm, z_hbm, out0, out1,
             acc, zbuf, sidx_a, didx_a, rows_a, sidx_b, didx_b, rows_b,
             isem_a, isem_b, gsem_a, gsem_b):
        c = lax.axis_index("c")
        s = lax.axis_index("s")
        base = s * rows_per_tile
        ebase = s * (chunks_per_tile * rpc)
        last = chunks_per_tile - 1

        pltpu.sync_copy(z_hbm, zbuf)
        _zero_acc_slice(acc, zbuf, base, rows_per_tile)
        plsc.subcore_barrier()

        def issue_idx(i, sidx, didx, isem):
            r0 = ebase + jnp.minimum(i, last) * rpc
            pltpu.async_copy(src_hbm.at[pl.ds(r0, rpc)], sidx, isem)
            pltpu.async_copy(dst_hbm.at[pl.ds(r0, rpc)], didx, isem)

        def wait_idx(sidx, didx, isem):
            pltpu.make_async_copy(src_hbm.at[pl.ds(0, rpc)], sidx, isem).wait()
            pltpu.make_async_copy(dst_hbm.at[pl.ds(0, rpc)], didx, isem).wait()

        def issue_gathers(t_hbm, sidx, rows, gsem):
            for j in range(rpc):
                pltpu.async_copy(t_hbm.at[sidx.at[j]], rows.at[j], gsem)

        def wait_gathers(t_hbm, sidx, rows, gsem):
            for j in range(rpc):
                pltpu.make_async_copy(t_hbm.at[sidx.at[j]], rows.at[j],
                                      gsem).wait()

        def issue_gathers_tc(sidx, rows, gsem):
            @pl.when(c == 0)
            def _():
                issue_gathers(t0_hbm, sidx, rows, gsem)

            @pl.when(c == 1)
            def _():
                issue_gathers(t1_hbm, sidx, rows, gsem)

        def scatter(didx, rows):
            for j in range(rpc):
                pltpu.sync_copy(rows.at[j], acc.at[didx.at[j]], add=True)

        issue_idx(0, sidx_a, didx_a, isem_a)
        wait_idx(sidx_a, didx_a, isem_a)
        issue_gathers_tc(sidx_a, rows_a, gsem_a)
        issue_idx(1, sidx_b, didx_b, isem_b)

        def half(i, cur, nxt):
            (sidx_s, didx_s, rows_s, isem_s, gsem_s) = cur
            (sidx_t, didx_t, rows_t, isem_t, gsem_t) = nxt
            wait_idx(sidx_t, didx_t, isem_t)
            wait_gathers(t0_hbm, sidx_s, rows_s, gsem_s)
            issue_gathers_tc(sidx_t, rows_t, gsem_t)
            scatter(didx_s, rows_s)
            issue_idx(i + 2, sidx_s, didx_s, isem_s)

        set_a = (sidx_a, didx_a, rows_a, isem_a, gsem_a)
        set_b = (sidx_b, didx_b, rows_b, isem_b, gsem_b)

        def pair(p, carry):
            half(2 * p, set_a, set_b)
            half(2 * p + 1, set_b, set_a)
            return carry

        lax.fori_loop(0, npairs, pair, 0)
        wait_idx(sidx_b, didx_b, isem_b)
        wait_gathers(t0_hbm, sidx_a, rows_a, gsem_a)
        plsc.subcore_barrier()

        @pl.when(c == 0)
        def _():
            pltpu.sync_copy(acc.at[pl.ds(base, rows_per_tile)],
                            out0.at[pl.ds(base, rows_per_tile)])

        @pl.when(c == 1)
        def _():
            pltpu.sync_copy(acc.at[pl.ds(base, rows_per_tile)],
                            out1.at[pl.ds(base, rows_per_tile)])

    return pl.kernel(
        body,
        out_type=[
            jax.ShapeDtypeStruct((n_acc, _HALF), jnp.float32),
            jax.ShapeDtypeStruct((n_acc, _HALF), jnp.float32),
        ],
        mesh=mesh,
        scratch_types=[
            pltpu.VMEM_SHARED((n_acc, _HALF), jnp.float32),
            pltpu.VMEM((128, 16), jnp.float32),
            pltpu.VMEM((rpc, _IB), jnp.int32),
            pltpu.VMEM((rpc, _IB), jnp.int32),
            pltpu.VMEM((rpc, _IB, _HALF), jnp.float32),
            pltpu.VMEM((rpc, _IB), jnp.int32),
            pltpu.VMEM((rpc, _IB), jnp.int32),
            pltpu.VMEM((rpc, _IB, _HALF), jnp.float32),
            pltpu.SemaphoreType.DMA,
            pltpu.SemaphoreType.DMA,
            pltpu.SemaphoreType.DMA,
            pltpu.SemaphoreType.DMA,
        ],
        compiler_params=pltpu.CompilerParams(use_tc_tiling_on_sc=False),
    )


def _make_deg_kernel(n_acc, rows_per_tile, chunks_per_range):
    mesh = plsc.VectorSubcoreMesh(core_axis_name="c", subcore_axis_name="s")
    rpc = _CH // _IB

    def body(dst_hbm, z_hbm, ones_hbm, out0, out1, acc, zbuf, ones_v, didx):
        c = lax.axis_index("c")
        s = lax.axis_index("s")
        base = s * rows_per_tile

        pltpu.sync_copy(z_hbm, zbuf)
        _zero_acc_slice(acc, zbuf, base, rows_per_tile)
        pltpu.sync_copy(ones_hbm, ones_v)
        plsc.subcore_barrier()

        def chunk(i, carry):
            w = c * _NT + s
            r0 = w * (chunks_per_range * rpc) + i * rpc
            pltpu.sync_copy(dst_hbm.at[pl.ds(r0, rpc)], didx)
            for j in range(rpc):
                pltpu.sync_copy(ones_v, acc.at[didx.at[j]], add=True)
            return carry

        lax.fori_loop(0, chunks_per_range, chunk, 0)
        plsc.subcore_barrier()

        @pl.when(c == 0)
        def _():
            pltpu.sync_copy(acc.at[pl.ds(base, rows_per_tile)],
                            out0.at[pl.ds(base, rows_per_tile)])

        @pl.when(c == 1)
        def _():
            pltpu.sync_copy(acc.at[pl.ds(base, rows_per_tile)],
                            out1.at[pl.ds(base, rows_per_tile)])

    return pl.kernel(
        body,
        out_type=[
            jax.ShapeDtypeStruct((n_acc, _HALF), jnp.float32),
            jax.ShapeDtypeStruct((n_acc, _HALF), jnp.float32),
        ],
        mesh=mesh,
        scratch_types=[
            pltpu.VMEM_SHARED((n_acc, _HALF), jnp.float32),
            pltpu.VMEM((128, 16), jnp.float32),
            pltpu.VMEM((_IB, _HALF), jnp.float32),
            pltpu.VMEM((rpc, _IB), jnp.int32),
        ],
        compiler_params=pltpu.CompilerParams(use_tc_tiling_on_sc=False),
    )


def kernel(x, edge_index, batch, W0, Ws, bs, Wc, bc):
    del bs, bc
    n = x.shape[0]
    e = edge_index.shape[1]

    rows_per_tile = (((n + _NT - 1) // _NT) + 7) // 8 * 8
    n_acc = rows_per_tile * _NT
    chunks_per_tile = (e + _NT * _CH - 1) // (_NT * _CH)
    e_pad = _NT * _CH * chunks_per_tile
    chunks_per_range = chunks_per_tile // 2

    src = edge_index[0]
    dst = edge_index[1]
    pad = e_pad - e
    if pad:
        src = jnp.concatenate([src, jnp.zeros((pad,), jnp.int32)])
        dst = jnp.concatenate([dst, jnp.full((pad,), n, jnp.int32)])
    src2d = src.reshape(-1, _IB)
    dst2d = dst.reshape(-1, _IB)

    agg = _make_agg_kernel(n_acc, rows_per_tile, chunks_per_tile)
    degk = _make_deg_kernel(n_acc, rows_per_tile, chunks_per_range)

    zeros2d = jnp.zeros((128, 16), jnp.float32)
    ones2d = jnp.ones((_IB, _HALF), jnp.float32)

    d0, d1 = degk(dst2d, zeros2d, ones2d)

    bn = 2000
    t0, t1 = _first_layer(x, W0, bn, n_acc)
    n_layers = Ws.shape[0] + 1
    a0 = a1 = None
    for i in range(n_layers):
        a0, a1 = agg(t0, t1, src2d, dst2d, zeros2d)
        if i + 1 < n_layers:
            t0, t1 = _mid_layer(a0, a1, d0, d1, Ws[i], n, bn, n_acc)

    batch2d = batch.reshape(n, 1)
    wct = Wc.reshape(-1, 1)
    return _final_layer(a0, a1, d0, d1, batch2d, wct, n, 1000)
